# Initial kernel scaffold; baseline (speedup 1.0000x reference)
import jax
import jax.numpy as jnp
from jax.experimental import pallas as pl


def kernel(ze, c_logits, books, log_param_q, is_train):
    raise NotImplementedError("write your pallas kernel here")


# speedup vs baseline: 2.2993x; 2.2993x over previous
import functools

import jax
import jax.numpy as jnp
from jax.experimental import pallas as pl
from jax.experimental.pallas import tpu as pltpu


def _vq_body(idx_ref, prec_ref, ze_ref, book_ref, prob_ref, logp_ref, zq_ref):
    prec = prec_ref[0]
    ze = ze_ref[0]
    book = book_ref[0]
    cross = jax.lax.dot_general(
        ze, book, (((1,), (1,)), ((), ())),
        preferred_element_type=jnp.float32)
    b_sq = jnp.sum(book * book, axis=1)
    t = (2.0 * prec) * cross - prec * b_sq[None, :]
    m = jnp.max(t, axis=1, keepdims=True)
    sh = t - m
    e = jnp.exp(sh)
    s = jnp.sum(e, axis=1, keepdims=True)
    prob_ref[0] = e / s
    logp_ref[0] = sh - jnp.log(s)
    am = jnp.argmax(t, axis=1)
    iota = jax.lax.broadcasted_iota(jnp.int32, t.shape, 1)
    enc = (iota == am[:, None]).astype(jnp.float32)
    zq_ref[0] = jax.lax.dot_general(
        enc, book, (((1,), (0,)), ((), ())),
        preferred_element_type=jnp.float32)


@jax.jit
def _vq(ze, c_logits, books, log_param_q):
    b, n, d = ze.shape
    n_books, K, _ = books.shape
    param_q = 1.0 + jnp.exp(log_param_q)
    precision_q = 0.5 / jnp.clip(param_q, 1e-10)
    idx = jnp.argmax(c_logits, axis=-1).astype(jnp.int32)
    prec_arr = jnp.reshape(precision_q.astype(jnp.float32), (1,))

    grid_spec = pltpu.PrefetchScalarGridSpec(
        num_scalar_prefetch=2,
        grid=(b,),
        in_specs=[
            pl.BlockSpec((1, n, d), lambda i, idx, prec: (i, 0, 0)),
            pl.BlockSpec((1, K, d), lambda i, idx, prec: (idx[i], 0, 0)),
        ],
        out_specs=[
            pl.BlockSpec((1, n, K), lambda i, idx, prec: (i, 0, 0)),
            pl.BlockSpec((1, n, K), lambda i, idx, prec: (i, 0, 0)),
            pl.BlockSpec((1, n, d), lambda i, idx, prec: (i, 0, 0)),
        ],
    )
    prob, log_prob, zq = pl.pallas_call(
        _vq_body,
        grid_spec=grid_spec,
        out_shape=[
            jax.ShapeDtypeStruct((b, n, K), jnp.float32),
            jax.ShapeDtypeStruct((b, n, K), jnp.float32),
            jax.ShapeDtypeStruct((b, n, d), jnp.float32),
        ],
    )(idx, prec_arr, ze, books)
    return zq, precision_q, prob, log_prob


def kernel(ze, c_logits, books, log_param_q, is_train):
    del is_train
    return _vq(ze, c_logits, books, log_param_q)
